# baseline (speedup 1.0000x reference)
import functools

import jax
import jax.numpy as jnp
from jax import lax
from jax.experimental import pallas as pl
from jax.experimental.pallas import tpu as pltpu
from jax.experimental.pallas import tpu_sc as plsc

_B = 16384
_D = 64
_NF = 5

_info = plsc.get_sparse_core_info()
_NC = _info.num_cores
_NS = _info.num_subcores
_NW = _NC * _NS
_BPW = _B // _NW

_NBUF = 3


def _build():
    mesh = plsc.VectorSubcoreMesh(core_axis_name="c", subcore_axis_name="s")

    @functools.partial(
        pl.kernel,
        mesh=mesh,
        out_type=jax.ShapeDtypeStruct((_B, _NF * _D), jnp.float32),
        scratch_types=[
            pltpu.VMEM((_NF, _BPW), jnp.int32),
            [pltpu.VMEM((_BPW, _D), jnp.float32) for _ in range(_NBUF)],
            pltpu.SemaphoreType.DMA,
            [pltpu.SemaphoreType.DMA for _ in range(_NBUF)],
            [pltpu.SemaphoreType.DMA for _ in range(_NBUF)],
        ],
        compiler_params=pltpu.CompilerParams(use_tc_tiling_on_sc=False),
    )
    def sc_kernel(t_idx, f_idx, st_idx, so_idx, y_idx,
                  t_tab, f_tab, st_tab, so_tab, y_tab,
                  out, idx_v, bufs, isem, gsems, ssems):
        wid = lax.axis_index("s") * _NC + lax.axis_index("c")
        base = wid * _BPW
        idx_arrays = (t_idx, f_idx, st_idx, so_idx, y_idx)
        tables = (t_tab, f_tab, st_tab, so_tab, y_tab)

        icopies = [
            pltpu.async_copy(idx_arrays[fi].at[pl.ds(base, _BPW)],
                             idx_v.at[fi], isem)
            for fi in range(_NF)
        ]
        for cp in icopies:
            cp.wait()

        gathers = {}
        scatters = {}

        def start_gather(fi):
            slot = fi % _NBUF
            gathers[fi] = pltpu.async_copy(
                tables[fi].at[idx_v.at[fi]], bufs[slot], gsems[slot])

        for fi in range(_NBUF):
            start_gather(fi)

        for fi in range(_NF):
            slot = fi % _NBUF
            gathers[fi].wait()
            scatters[fi] = pltpu.async_copy(
                bufs[slot],
                out.at[pl.ds(base, _BPW), pl.ds(fi * _D, _D)],
                ssems[slot])
            nxt = fi + _NBUF
            if nxt < _NF:
                scatters[fi].wait()
                start_gather(nxt)

        for fi in range(_NF):
            if fi + _NBUF >= _NF:
                scatters[fi].wait()

    return sc_kernel


_sc_kernel = _build()


@jax.jit
def kernel(title_idx, format_idx, studio_idx, source_idx, year_idx,
           title_table, format_table, studio_table, source_table, year_table):
    return _sc_kernel(title_idx, format_idx, studio_idx, source_idx, year_idx,
                      title_table, format_table, studio_table, source_table,
                      year_table)


# speedup vs baseline: 1.4276x; 1.4276x over previous
import functools

import jax
import jax.numpy as jnp
from jax import lax
from jax.experimental import pallas as pl
from jax.experimental.pallas import tpu as pltpu
from jax.experimental.pallas import tpu_sc as plsc

_B = 16384
_D = 64
_NF = 5
_BLK = 128

_info = plsc.get_sparse_core_info()
_NC = _info.num_cores
_NS = _info.num_subcores
_NW = _NC * _NS
_BPW = _B // _NW
_NBLK = _BPW // _BLK

_SMALL = ((1, 11), (3, 17), (4, 81))
_STREAMED = (0, 2)


def _build():
    mesh = plsc.VectorSubcoreMesh(core_axis_name="c", subcore_axis_name="s")

    @functools.partial(
        pl.kernel,
        mesh=mesh,
        out_type=jax.ShapeDtypeStruct((_B, _NF * _D), jnp.float32),
        scratch_types=[
            pltpu.VMEM((5, _BPW), jnp.int32),
            pltpu.VMEM((11, _D), jnp.float32),
            pltpu.VMEM((17, _D), jnp.float32),
            pltpu.VMEM((81, _D), jnp.float32),
            [pltpu.VMEM((_BPW, _D), jnp.float32) for _ in range(2)],
            [[pltpu.VMEM((_BLK, _D), jnp.float32) for _ in range(3)]
             for _ in range(2)],
            pltpu.SemaphoreType.DMA,
            [pltpu.SemaphoreType.DMA for _ in range(2)],
            [pltpu.SemaphoreType.DMA for _ in range(2)],
            [pltpu.SemaphoreType.DMA for _ in range(2)],
        ],
        compiler_params=pltpu.CompilerParams(use_tc_tiling_on_sc=False),
    )
    def sc_kernel(t_idx, f_idx, st_idx, so_idx, y_idx,
                  t_tab, f_tab, st_tab, so_tab, y_tab,
                  out, idx_v, fv, sov, yv, gbufs, ebufs,
                  isem, gsems, wsems, esems):
        wid = lax.axis_index("s") * _NC + lax.axis_index("c")
        base = wid * _BPW
        small_tabs = (fv, sov, yv)

        stages = [
            pltpu.async_copy(t_idx.at[pl.ds(base, _BPW)], idx_v.at[0], isem),
            pltpu.async_copy(st_idx.at[pl.ds(base, _BPW)], idx_v.at[1], isem),
            pltpu.async_copy(f_idx.at[pl.ds(base, _BPW)], idx_v.at[2], isem),
            pltpu.async_copy(so_idx.at[pl.ds(base, _BPW)], idx_v.at[3], isem),
            pltpu.async_copy(y_idx.at[pl.ds(base, _BPW)], idx_v.at[4], isem),
            pltpu.async_copy(f_tab, fv, isem),
            pltpu.async_copy(so_tab, sov, isem),
            pltpu.async_copy(y_tab, yv, isem),
        ]
        stages[0].wait()
        stages[1].wait()
        g_title = pltpu.async_copy(t_tab.at[idx_v.at[0]], gbufs[0], gsems[0])
        g_studio = pltpu.async_copy(st_tab.at[idx_v.at[1]], gbufs[1], gsems[1])
        for cp in stages[2:]:
            cp.wait()

        ewrites = {}
        for b in range(_NBLK):
            slot = b % 2
            if b >= 2:
                for w in ewrites[b - 2]:
                    w.wait()

            def body(r, carry, b=b, slot=slot):
                for k, (fi, _) in enumerate(_SMALL):
                    row = idx_v[2 + k, pl.ds(b * _BLK + r, 1)][0]
                    for c in range(_D // 16):
                        ebufs[slot][k][r, pl.ds(c * 16, 16)] = (
                            small_tabs[k][row, pl.ds(c * 16, 16)])
                return carry

            lax.fori_loop(0, _BLK, body, 0)
            ewrites[b] = [
                pltpu.async_copy(
                    ebufs[slot][k],
                    out.at[pl.ds(base + b * _BLK, _BLK),
                           pl.ds(fi * _D, _D)],
                    esems[slot])
                for k, (fi, _) in enumerate(_SMALL)
            ]

        g_title.wait()
        w_title = pltpu.async_copy(
            gbufs[0], out.at[pl.ds(base, _BPW), pl.ds(0, _D)], wsems[0])
        g_studio.wait()
        w_studio = pltpu.async_copy(
            gbufs[1], out.at[pl.ds(base, _BPW), pl.ds(2 * _D, _D)], wsems[1])

        for b in (_NBLK - 2, _NBLK - 1):
            for w in ewrites[b]:
                w.wait()
        w_title.wait()
        w_studio.wait()

    return sc_kernel


_sc_kernel = _build()


@jax.jit
def kernel(title_idx, format_idx, studio_idx, source_idx, year_idx,
           title_table, format_table, studio_table, source_table, year_table):
    return _sc_kernel(title_idx, format_idx, studio_idx, source_idx, year_idx,
                      title_table, format_table, studio_table, source_table,
                      year_table)
